# Initial kernel scaffold; baseline (speedup 1.0000x reference)
import functools

import jax
import jax.numpy as jnp
from jax import lax
from jax.experimental import pallas as pl
from jax.experimental.pallas import tpu as pltpu
from jax.experimental.pallas import tpu_sc as plsc

_LANES = 16
_TILES = 16
_CORES = 2
_C = 128


def _spmm_pass(tab_h, colv, rowv, valv, gbuf, acc, sem, nch, fh):
    def chunk_body(k, carry):
        pltpu.async_copy(tab_h.at[colv.at[k]], gbuf, sem).wait()

        def group_body(g, c2):
            vg = valv[k, pl.ds(g * _LANES, _LANES)]
            for e in range(_LANES):
                sc = vg[e]
                r = g * _LANES + e
                for j in range(fh // _LANES):
                    sl = pl.ds(j * _LANES, _LANES)
                    gbuf[r, sl] = gbuf[r, sl] * sc
            return c2

        lax.fori_loop(0, _C // _LANES, group_body, 0)

        pltpu.sync_copy(gbuf, acc.at[rowv.at[k]], add=True)
        return carry

    lax.fori_loop(0, nch, chunk_body, 0)


def _sc_cheb_spmm(cols3, rows3, vals3, xs_flat, np_pad, fh, nch):
    stripe = np_pad // _TILES
    mesh = plsc.VectorSubcoreMesh(
        core_axis_name="c", subcore_axis_name="s",
        num_cores=_CORES, num_subcores=_TILES,
    )

    @functools.partial(
        pl.kernel,
        out_type=[
            jax.ShapeDtypeStruct((_CORES * np_pad, fh), jnp.float32),
            jax.ShapeDtypeStruct((_CORES * np_pad, fh), jnp.float32),
        ],
        mesh=mesh,
        scratch_types=[
            pltpu.VMEM((nch, _C), jnp.int32),
            pltpu.VMEM((nch, _C), jnp.int32),
            pltpu.VMEM((nch, _C), jnp.float32),
            pltpu.VMEM((_C, fh), jnp.float32),
            pltpu.VMEM_SHARED((np_pad, fh), jnp.float32),
            pltpu.VMEM_SHARED((np_pad, fh), jnp.float32),
            pltpu.SemaphoreType.DMA,
        ],
    )
    def body(cols_h, rows_h, vals_h, xs_h, s1_h, s2_h,
             colv, rowv, valv, gbuf, acc1, acc2, sem):
        c = lax.axis_index("c")
        s = lax.axis_index("s")
        base = s * stripe

        pltpu.sync_copy(cols_h.at[s], colv)
        pltpu.sync_copy(rows_h.at[s], rowv)
        pltpu.sync_copy(vals_h.at[s], valv)

        def zero_row(i, carry):
            for j in range(fh // _LANES):
                gbuf[i, pl.ds(j * _LANES, _LANES)] = jnp.zeros(
                    (_LANES,), jnp.float32)
            return carry

        lax.fori_loop(0, _C, zero_row, 0)
        for kk in range(stripe // _C):
            pltpu.sync_copy(gbuf, acc1.at[pl.ds(base + kk * _C, _C)])
            pltpu.sync_copy(gbuf, acc2.at[pl.ds(base + kk * _C, _C)])

        off = jnp.full((_LANES,), c * np_pad, jnp.int32)

        def off_body(i, carry):
            for j in range(_C // _LANES):
                sl = pl.ds(j * _LANES, _LANES)
                colv[i, sl] = colv[i, sl] + off
            return carry

        lax.fori_loop(0, nch, off_body, 0)

        plsc.subcore_barrier()

        _spmm_pass(xs_h, colv, rowv, valv, gbuf, acc1, sem, nch, fh)
        plsc.subcore_barrier()

        for kk in range(stripe // _C):
            pltpu.sync_copy(
                acc1.at[pl.ds(base + kk * _C, _C)],
                s1_h.at[pl.ds(c * np_pad + base + kk * _C, _C)])
        plsc.subcore_barrier()

        _spmm_pass(s1_h, colv, rowv, valv, gbuf, acc2, sem, nch, fh)
        plsc.subcore_barrier()

        for kk in range(stripe // _C):
            pltpu.sync_copy(
                acc2.at[pl.ds(base + kk * _C, _C)],
                s2_h.at[pl.ds(c * np_pad + base + kk * _C, _C)])

    return body(cols3, rows3, vals3, xs_flat)


def _combine_body(x_ref, s1_ref, s2_ref, w_ref, b_ref, o_ref):
    acc = jnp.dot(x_ref[0], w_ref[0], preferred_element_type=jnp.float32)
    acc += jnp.dot(x_ref[1], w_ref[1], preferred_element_type=jnp.float32)
    acc += jnp.dot(s1_ref[0], w_ref[2], preferred_element_type=jnp.float32)
    acc += jnp.dot(s1_ref[1], w_ref[3], preferred_element_type=jnp.float32)
    acc += jnp.dot(s2_ref[0], w_ref[4], preferred_element_type=jnp.float32)
    acc += jnp.dot(s2_ref[1], w_ref[5], preferred_element_type=jnp.float32)
    o_ref[...] = acc + b_ref[...]


def _tc_combine(xs3, s1s, s2s, wb, bb, np_pad, fh, outf, bm):
    grid = (np_pad // bm,)
    return pl.pallas_call(
        _combine_body,
        grid=grid,
        in_specs=[
            pl.BlockSpec((2, bm, fh), lambda i: (0, i, 0)),
            pl.BlockSpec((2, bm, fh), lambda i: (0, i, 0)),
            pl.BlockSpec((2, bm, fh), lambda i: (0, i, 0)),
            pl.BlockSpec((6, fh, outf), lambda i: (0, 0, 0)),
            pl.BlockSpec((1, outf), lambda i: (0, 0)),
        ],
        out_specs=pl.BlockSpec((bm, outf), lambda i: (i, 0)),
        out_shape=jax.ShapeDtypeStruct((np_pad, outf), jnp.float32),
    )(xs3, s1s, s2s, wb, bb)


def kernel(x, laplacian_indices, laplacian_values, W, b):
    n, f = x.shape
    e = laplacian_values.shape[0]
    outf = W.shape[0]
    k = W.shape[1] // f
    assert k == 3 and f % (2 * _LANES) == 0
    fh = f // 2

    stripe = -(-n // (_TILES * _C)) * _C
    np_pad = _TILES * stripe
    ept = -(-e // (_TILES * _C)) * _C
    nch = ept // _C
    ep = _TILES * ept

    rows = jnp.pad(laplacian_indices[0], (0, ep - e)).reshape(_TILES, nch, _C)
    cols = jnp.pad(laplacian_indices[1], (0, ep - e)).reshape(_TILES, nch, _C)
    vals = jnp.pad(laplacian_values, (0, ep - e)).reshape(_TILES, nch, _C)

    xp = jnp.pad(x, ((0, np_pad - n), (0, 0)))
    xs_flat = jnp.concatenate([xp[:, :fh], xp[:, fh:]], axis=0)

    w0 = W[:, 0::3]
    w1 = W[:, 1::3]
    w2 = W[:, 2::3]
    a = (w0 - w2).T
    bt = w1.T
    ct = 2.0 * w2.T
    wb = jnp.stack([a[:fh], a[fh:], bt[:fh], bt[fh:], ct[:fh], ct[fh:]])

    s1_flat, s2_flat = _sc_cheb_spmm(cols, rows, vals, xs_flat,
                                     np_pad, fh, nch)

    xs3 = xs_flat.reshape(2, np_pad, fh)
    s1s = s1_flat.reshape(2, np_pad, fh)
    s2s = s2_flat.reshape(2, np_pad, fh)

    outp = _tc_combine(xs3, s1s, s2s, wb, b.reshape(1, outf),
                       np_pad, fh, outf, bm=640)
    return outp[:n]


# speedup vs baseline: 4.5031x; 4.5031x over previous
import functools

import jax
import jax.numpy as jnp
from jax import lax
from jax.experimental import pallas as pl
from jax.experimental.pallas import tpu as pltpu
from jax.experimental.pallas import tpu_sc as plsc

_LANES = 16
_TILES = 16
_CORES = 2
_C = 128


def _spmm_pass(tab_h, colv, rowv, valv, gbuf, acc, sem, nch, fh):
    def chunk_body(k, carry):
        pltpu.async_copy(tab_h.at[colv.at[k]], gbuf, sem).wait()

        def group_body(g, c2):
            vg = valv[k, pl.ds(g * _LANES, _LANES)]
            for e in range(_LANES):
                sc = vg[e]
                r = g * _LANES + e
                for j in range(fh // _LANES):
                    sl = pl.ds(j * _LANES, _LANES)
                    gbuf[r, sl] = gbuf[r, sl] * sc
            return c2

        lax.fori_loop(0, _C // _LANES, group_body, 0)

        pltpu.sync_copy(gbuf, acc.at[rowv.at[k]], add=True)
        return carry

    lax.fori_loop(0, nch, chunk_body, 0)


def _sc_cheb_spmm(cols3, rows3, vals3, xs_flat, np_pad, fh, nch):
    stripe = np_pad // _TILES
    mesh = plsc.VectorSubcoreMesh(
        core_axis_name="c", subcore_axis_name="s",
        num_cores=_CORES, num_subcores=_TILES,
    )

    @functools.partial(
        pl.kernel,
        out_type=[
            jax.ShapeDtypeStruct((_CORES * np_pad, fh), jnp.float32),
            jax.ShapeDtypeStruct((_CORES * np_pad, fh), jnp.float32),
        ],
        mesh=mesh,
        compiler_params=pltpu.CompilerParams(use_tc_tiling_on_sc=False),
        scratch_types=[
            pltpu.VMEM((nch, _C), jnp.int32),
            pltpu.VMEM((nch, _C), jnp.int32),
            pltpu.VMEM((nch, _C), jnp.float32),
            pltpu.VMEM((_C, fh), jnp.float32),
            pltpu.VMEM_SHARED((np_pad, fh), jnp.float32),
            pltpu.SemaphoreType.DMA,
        ],
    )
    def body(cols_h, rows_h, vals_h, xs_h, s1_h, s2_h,
             colv, rowv, valv, gbuf, acc, sem):
        c = lax.axis_index("c")
        s = lax.axis_index("s")
        base = s * stripe

        pltpu.sync_copy(cols_h.at[s], colv)
        pltpu.sync_copy(rows_h.at[s], rowv)
        pltpu.sync_copy(vals_h.at[s], valv)

        def zero_acc_stripe():
            def zero_row(i, carry):
                for j in range(fh // _LANES):
                    gbuf[i, pl.ds(j * _LANES, _LANES)] = jnp.zeros(
                        (_LANES,), jnp.float32)
                return carry

            lax.fori_loop(0, _C, zero_row, 0)
            for kk in range(stripe // _C):
                pltpu.sync_copy(gbuf, acc.at[pl.ds(base + kk * _C, _C)])

        zero_acc_stripe()

        off = jnp.full((_LANES,), c * np_pad, jnp.int32)

        def off_body(i, carry):
            for j in range(_C // _LANES):
                sl = pl.ds(j * _LANES, _LANES)
                colv[i, sl] = colv[i, sl] + off
            return carry

        lax.fori_loop(0, nch, off_body, 0)

        plsc.subcore_barrier()

        _spmm_pass(xs_h, colv, rowv, valv, gbuf, acc, sem, nch, fh)
        plsc.subcore_barrier()

        for kk in range(stripe // _C):
            pltpu.sync_copy(
                acc.at[pl.ds(base + kk * _C, _C)],
                s1_h.at[pl.ds(c * np_pad + base + kk * _C, _C)])
        zero_acc_stripe()
        plsc.subcore_barrier()

        _spmm_pass(s1_h, colv, rowv, valv, gbuf, acc, sem, nch, fh)
        plsc.subcore_barrier()

        for kk in range(stripe // _C):
            pltpu.sync_copy(
                acc.at[pl.ds(base + kk * _C, _C)],
                s2_h.at[pl.ds(c * np_pad + base + kk * _C, _C)])

    return body(cols3, rows3, vals3, xs_flat)


def _combine_body(x_ref, s1_ref, s2_ref, w_ref, b_ref, o_ref):
    acc = jnp.dot(x_ref[0], w_ref[0], preferred_element_type=jnp.float32)
    acc += jnp.dot(x_ref[1], w_ref[1], preferred_element_type=jnp.float32)
    acc += jnp.dot(s1_ref[0], w_ref[2], preferred_element_type=jnp.float32)
    acc += jnp.dot(s1_ref[1], w_ref[3], preferred_element_type=jnp.float32)
    acc += jnp.dot(s2_ref[0], w_ref[4], preferred_element_type=jnp.float32)
    acc += jnp.dot(s2_ref[1], w_ref[5], preferred_element_type=jnp.float32)
    o_ref[...] = acc + b_ref[...]


def _tc_combine(xs3, s1s, s2s, wb, bb, np_pad, fh, outf, bm):
    grid = (np_pad // bm,)
    return pl.pallas_call(
        _combine_body,
        grid=grid,
        in_specs=[
            pl.BlockSpec((2, bm, fh), lambda i: (0, i, 0)),
            pl.BlockSpec((2, bm, fh), lambda i: (0, i, 0)),
            pl.BlockSpec((2, bm, fh), lambda i: (0, i, 0)),
            pl.BlockSpec((6, fh, outf), lambda i: (0, 0, 0)),
            pl.BlockSpec((1, outf), lambda i: (0, 0)),
        ],
        out_specs=pl.BlockSpec((bm, outf), lambda i: (i, 0)),
        out_shape=jax.ShapeDtypeStruct((np_pad, outf), jnp.float32),
    )(xs3, s1s, s2s, wb, bb)


def kernel(x, laplacian_indices, laplacian_values, W, b):
    n, f = x.shape
    e = laplacian_values.shape[0]
    outf = W.shape[0]
    k = W.shape[1] // f
    assert k == 3 and f % (2 * _LANES) == 0
    fh = f // 2

    stripe = -(-n // (_TILES * _C)) * _C
    np_pad = _TILES * stripe
    ept = -(-e // (_TILES * _C)) * _C
    nch = ept // _C
    ep = _TILES * ept

    rows = jnp.pad(laplacian_indices[0], (0, ep - e)).reshape(_TILES, nch, _C)
    cols = jnp.pad(laplacian_indices[1], (0, ep - e)).reshape(_TILES, nch, _C)
    vals = jnp.pad(laplacian_values, (0, ep - e)).reshape(_TILES, nch, _C)

    xp = jnp.pad(x, ((0, np_pad - n), (0, 0)))
    xs_flat = jnp.concatenate([xp[:, :fh], xp[:, fh:]], axis=0)

    w0 = W[:, 0::3]
    w1 = W[:, 1::3]
    w2 = W[:, 2::3]
    a = (w0 - w2).T
    bt = w1.T
    ct = 2.0 * w2.T
    wb = jnp.stack([a[:fh], a[fh:], bt[:fh], bt[fh:], ct[:fh], ct[fh:]])

    s1_flat, s2_flat = _sc_cheb_spmm(cols, rows, vals, xs_flat,
                                     np_pad, fh, nch)

    xs3 = xs_flat.reshape(2, np_pad, fh)
    s1s = s1_flat.reshape(2, np_pad, fh)
    s2s = s2_flat.reshape(2, np_pad, fh)

    outp = _tc_combine(xs3, s1s, s2s, wb, b.reshape(1, outf),
                       np_pad, fh, outf, bm=640)
    return outp[:n]
